# Initial kernel scaffold; baseline (speedup 1.0000x reference)
import jax
import jax.numpy as jnp
from jax.experimental import pallas as pl


def kernel(x, adj, W0, b0, gamma0, beta0, W1, b1, gamma1, beta1, W2, b2, gamma2, beta2):
    raise NotImplementedError("write your pallas kernel here")


# speedup vs baseline: 3.7958x; 3.7958x over previous
import jax
import jax.numpy as jnp
from jax.experimental import pallas as pl
from jax.experimental.pallas import tpu as pltpu

B, N, C = 8, 2048, 128
BLK = 512
NBLK = N // BLK
EPS = 1e-5


def _layer_body(adj_ref, h_ref, w_ref, bias_ref, r_ref, sum_ref, sq_ref, z_ref):
    b = pl.program_id(0)
    i = pl.program_id(1)

    @pl.when(i == 0)
    def _():
        z_ref[...] = (
            jnp.dot(h_ref[0], w_ref[...], preferred_element_type=jnp.float32)
            + bias_ref[...]
        )

    rows = i * BLK + jax.lax.broadcasted_iota(jnp.int32, (BLK, N), 0)
    cols = jax.lax.broadcasted_iota(jnp.int32, (BLK, N), 1)
    a = jnp.where(rows == cols, 1.0, adj_ref[0])

    m = jnp.dot(a, z_ref[...], preferred_element_type=jnp.float32)
    r = jnp.maximum(m, 0.0)
    r_ref[0] = r

    @pl.when((b == 0) & (i == 0))
    def _():
        sum_ref[...] = jnp.zeros_like(sum_ref)
        sq_ref[...] = jnp.zeros_like(sq_ref)

    sum_ref[...] += jnp.sum(r, axis=0, keepdims=True)
    sq_ref[...] += jnp.sum(r * r, axis=0, keepdims=True)


def _layer(adj, h, w_eff, b_eff):
    return pl.pallas_call(
        _layer_body,
        grid=(B, NBLK),
        in_specs=[
            pl.BlockSpec((1, BLK, N), lambda b, i: (b, i, 0)),
            pl.BlockSpec((1, N, C), lambda b, i: (b, 0, 0)),
            pl.BlockSpec((C, C), lambda b, i: (0, 0)),
            pl.BlockSpec((1, C), lambda b, i: (0, 0)),
        ],
        out_specs=[
            pl.BlockSpec((1, BLK, C), lambda b, i: (b, i, 0)),
            pl.BlockSpec((1, C), lambda b, i: (0, 0)),
            pl.BlockSpec((1, C), lambda b, i: (0, 0)),
        ],
        out_shape=[
            jax.ShapeDtypeStruct((B, N, C), jnp.float32),
            jax.ShapeDtypeStruct((1, C), jnp.float32),
            jax.ShapeDtypeStruct((1, C), jnp.float32),
        ],
        scratch_shapes=[pltpu.VMEM((N, C), jnp.float32)],
    )(adj, h, w_eff, b_eff.reshape(1, C))


def _affine_body(r_ref, s_ref, t_ref, o_ref):
    o_ref[...] = r_ref[...] * s_ref[...] + t_ref[...]


def _final_affine(r, s, t):
    rf = r.reshape(B * N, C)
    out = pl.pallas_call(
        _affine_body,
        grid=(B * N // 2048,),
        in_specs=[
            pl.BlockSpec((2048, C), lambda i: (i, 0)),
            pl.BlockSpec((1, C), lambda i: (0, 0)),
            pl.BlockSpec((1, C), lambda i: (0, 0)),
        ],
        out_specs=pl.BlockSpec((2048, C), lambda i: (i, 0)),
        out_shape=jax.ShapeDtypeStruct((B * N, C), jnp.float32),
    )(rf, s.reshape(1, C), t.reshape(1, C))
    return out.reshape(B, N, C)


def kernel(x, adj, W0, b0, gamma0, beta0, W1, b1, gamma1, beta1, W2, b2, gamma2, beta2):
    Ws = [W0, W1, W2]
    bs = [b0, b1, b2]
    gammas = [gamma0, gamma1, gamma2]
    betas = [beta0, beta1, beta2]

    h = x
    s = t = None
    cnt = float(B * N)
    for l in range(3):
        if l == 0:
            w_eff, b_eff = Ws[0], bs[0]
        else:
            w_eff = s[:, None] * Ws[l]
            b_eff = t @ Ws[l] + bs[l]
        h, sm, sq = _layer(adj, h, w_eff, b_eff)
        mean = sm[0] / cnt
        var = sq[0] / cnt - mean * mean
        s = gammas[l] * jax.lax.rsqrt(var + EPS)
        t = betas[l] - mean * s
    return _final_affine(h, s, t)
